# baseline (speedup 1.0000x reference)
import functools
import math

import jax
import jax.numpy as jnp
from jax import lax
from jax.experimental import pallas as pl
from jax.experimental.pallas import tpu as pltpu
from jax.experimental.pallas import tpu_sc as plsc

D_MODEL = 64
SCALE = math.sqrt(D_MODEL)

_INFO = plsc.get_sparse_core_info()
_NC, _NS, _L = _INFO.num_cores, _INFO.num_subcores, _INFO.num_lanes
_NW = _NC * _NS

PANEL = 512


def _make_build_table(V: int):
    n_full = V // PANEL
    tail = V - n_full * PANEL
    mesh = plsc.VectorSubcoreMesh(core_axis_name="c", subcore_axis_name="s")

    @functools.partial(
        pl.kernel,
        mesh=mesh,
        out_type=jax.ShapeDtypeStruct((V // 2, 2 * D_MODEL), jnp.float32),
        scratch_types=[
            pltpu.VMEM((D_MODEL, PANEL), jnp.float32),
            pltpu.VMEM((PANEL // 2, 2 * D_MODEL), jnp.float32),
        ],
        compiler_params=pltpu.CompilerParams(use_tc_tiling_on_sc=True,
                                             needs_layout_passes=False),
    )
    def build(lutt_hbm, tailt_hbm, p_hbm, a_v, b_v):
        wid = lax.axis_index("s") * _NC + lax.axis_index("c")

        def do_panel(p, width):
            c0 = pl.multiple_of(p * PANEL, PANEL)
            pltpu.sync_copy(lutt_hbm.at[:, pl.ds(c0, width)],
                            a_v.at[:, pl.ds(0, width)])

            @pl.loop(0, width)
            def _col(c):
                h = lax.rem(c, 2)
                r2 = lax.div(c, 2)
                base = lax.mul(h, D_MODEL)
                for j in range(D_MODEL // _L):
                    rows = lax.iota(jnp.int32, _L) + jnp.int32(j * _L)
                    cols = jnp.zeros((_L,), jnp.int32) + c
                    v = plsc.load_gather(a_v, [rows, cols])
                    b_v[r2, pl.ds(base + j * _L, _L)] = v * jnp.float32(SCALE)

            pltpu.sync_copy(b_v.at[pl.ds(0, width // 2)],
                            p_hbm.at[pl.ds(pl.multiple_of(p * (PANEL // 2),
                                                          PANEL // 2),
                                           width // 2)])

        @pl.loop(0, n_full)
        def _panel(p):
            @pl.when(lax.rem(p, _NW) == wid)
            def _mine():
                do_panel(p, PANEL)

        if tail:
            @pl.when(wid == 0)
            def _tail():
                pltpu.sync_copy(tailt_hbm, a_v.at[:, pl.ds(0, 128)])

                @pl.loop(0, tail)
                def _col(c):
                    h = lax.rem(c, 2)
                    r2 = lax.div(c, 2)
                    base = lax.mul(h, D_MODEL)
                    for j in range(D_MODEL // _L):
                        rows = lax.iota(jnp.int32, _L) + jnp.int32(j * _L)
                        cols = jnp.zeros((_L,), jnp.int32) + (c + (128 - tail))
                        v = plsc.load_gather(a_v, [rows, cols])
                        b_v[r2, pl.ds(base + j * _L, _L)] = (
                            v * jnp.float32(SCALE))

                pltpu.sync_copy(
                    b_v.at[pl.ds(0, tail // 2)],
                    p_hbm.at[pl.ds(pl.multiple_of(n_full * (PANEL // 2),
                                                  PANEL // 2), tail // 2)])

    return build


def _make_lookup(S: int, T: int, V: int):
    s_per_w = S // _NW
    mesh = plsc.VectorSubcoreMesh(core_axis_name="c", subcore_axis_name="s")

    @functools.partial(
        pl.kernel,
        mesh=mesh,
        out_type=jax.ShapeDtypeStruct((T, D_MODEL, S), jnp.float32),
        scratch_types=[
            pltpu.VMEM((T, s_per_w), jnp.int32),
            pltpu.VMEM((2, s_per_w), jnp.int32),
            pltpu.VMEM((2, s_per_w), jnp.int32),
            pltpu.VMEM((2, s_per_w, 2 * D_MODEL), jnp.float32),
            pltpu.VMEM((2, D_MODEL, s_per_w), jnp.float32),
            pltpu.SemaphoreType.DMA,
            pltpu.SemaphoreType.DMA,
        ],
        compiler_params=pltpu.CompilerParams(use_tc_tiling_on_sc=True,
                                             needs_layout_passes=False),
    )
    def lookup(xt_hbm, p_hbm, outt_hbm, xb_v, ih_v, off_v, g_v, o_v,
               sem_g, sem_s):
        wid = lax.axis_index("s") * _NC + lax.axis_index("c")
        s0 = pl.multiple_of(wid * s_per_w, s_per_w)
        pltpu.sync_copy(xt_hbm.at[:, pl.ds(s0, s_per_w)], xb_v)

        def prep(t, buf):
            @pl.loop(0, s_per_w // _L, unroll=8)
            def _p(i):
                sl = pl.ds(i * _L, _L)
                idx = xb_v[t, sl]
                ih_v[buf, sl] = lax.shift_right_logical(idx, 1)
                off_v[buf, sl] = lax.shift_left(
                    lax.bitwise_and(idx, jnp.int32(1)), 6)

        def gather(buf):
            return pltpu.make_async_copy(
                p_hbm.at[ih_v.at[buf]], g_v.at[buf], sem_g)

        def store(t, buf):
            return pltpu.make_async_copy(
                o_v.at[buf], outt_hbm.at[t].at[:, pl.ds(s0, s_per_w)], sem_s)

        def transpose_slab(buf):
            g = g_v.at[buf]
            for k in range(s_per_w // _L):
                rows = lax.iota(jnp.int32, _L) + jnp.int32(k * _L)
                cols0 = off_v[buf, pl.ds(k * _L, _L)]

                def body(d, cols):
                    o_v[buf, d, pl.ds(k * _L, _L)] = plsc.load_gather(
                        g, [rows, cols])
                    return cols + 1

                lax.fori_loop(0, D_MODEL, body, cols0, unroll=8)

        prep(0, 0)
        gather(0).start()

        @pl.loop(0, T, step=2)
        def _t0(t0):
            for b in range(2):
                t = t0 + b
                nb = 1 - b

                @pl.when(t + 1 < T)
                def _fire_next():
                    prep(t + 1, nb)
                    gather(nb).start()

                gather(b).wait()

                @pl.when(t >= 2)
                def _drain_store():
                    store(t - 2, b).wait()

                transpose_slab(b)
                store(t, b).start()

        store(T - 2, 0).wait()
        store(T - 1, 1).wait()

    return lookup


def kernel(x, lut):
    S, T = x.shape
    V = lut.shape[0]
    lutt = lut.T
    tail = V - (V // PANEL) * PANEL
    p = _make_build_table(V)(lutt, lax.slice(lutt, (0, V - 128), (D_MODEL, V)))
    outt = _make_lookup(S, T, V)(x.T, p)
    return jnp.transpose(outt, (2, 0, 1))


# speedup vs baseline: 2.5338x; 2.5338x over previous
import functools
import math

import jax
import jax.numpy as jnp
from jax import lax
from jax.experimental import pallas as pl
from jax.experimental.pallas import tpu as pltpu
from jax.experimental.pallas import tpu_sc as plsc

D_MODEL = 64
SCALE = math.sqrt(D_MODEL)

_INFO = plsc.get_sparse_core_info()
_NC, _NS, _L = _INFO.num_cores, _INFO.num_subcores, _INFO.num_lanes
_NW = _NC * _NS

CHUNK = 512
SUBGATHER = 128


def _make_emb(B: int):
    b_per_w = B // _NW
    nchunks = b_per_w // CHUNK
    n_sub = CHUNK // SUBGATHER
    vregs_per_row = D_MODEL // _L
    assert nchunks % 2 == 0 and nchunks >= 2

    mesh = plsc.VectorSubcoreMesh(core_axis_name="c", subcore_axis_name="s")

    @functools.partial(
        pl.kernel,
        mesh=mesh,
        out_type=jax.ShapeDtypeStruct((B, D_MODEL), jnp.float32),
        scratch_types=[
            pltpu.VMEM((b_per_w,), jnp.int32),
            pltpu.VMEM((2, CHUNK, D_MODEL), jnp.float32),
            pltpu.SemaphoreType.DMA,
            pltpu.SemaphoreType.DMA,
        ],
        compiler_params=pltpu.CompilerParams(use_tc_tiling_on_sc=False),
    )
    def emb(x_hbm, lut_hbm, out_hbm, idx_v, rows_v, sem_g, sem_s):
        wid = lax.axis_index("s") * _NC + lax.axis_index("c")
        base = wid * b_per_w
        pltpu.sync_copy(x_hbm.at[pl.ds(base, b_per_w)], idx_v)

        def fire_gather(ci, buf):
            for k in range(n_sub):
                pltpu.async_copy(
                    lut_hbm.at[idx_v.at[pl.ds(ci * CHUNK + k * SUBGATHER,
                                              SUBGATHER)]],
                    rows_v.at[buf].at[pl.ds(k * SUBGATHER, SUBGATHER)],
                    sem_g,
                )

        def wait_gather(ci, buf):
            for k in range(n_sub):
                pltpu.make_async_copy(
                    lut_hbm.at[idx_v.at[pl.ds(ci * CHUNK + k * SUBGATHER,
                                              SUBGATHER)]],
                    rows_v.at[buf].at[pl.ds(k * SUBGATHER, SUBGATHER)],
                    sem_g,
                ).wait()

        fire_gather(0, 0)

        @pl.loop(0, nchunks, step=2)
        def _outer(ci0):
            for b in range(2):
                ci = ci0 + b
                nb = 1 - b

                @pl.when(ci >= 1)
                def _wait_prev_store():
                    pltpu.make_async_copy(
                        rows_v.at[nb],
                        out_hbm.at[pl.ds(base + (ci - 1) * CHUNK, CHUNK)],
                        sem_s,
                    ).wait()

                @pl.when(ci + 1 < nchunks)
                def _fire_next_gather():
                    fire_gather(ci + 1, nb)

                wait_gather(ci, b)

                @pl.loop(0, CHUNK, unroll=8)
                def _scale(r):
                    for j in range(vregs_per_row):
                        sl = pl.ds(j * _L, _L)
                        rows_v[b, r, sl] = rows_v[b, r, sl] * jnp.float32(SCALE)

                pltpu.async_copy(
                    rows_v.at[b],
                    out_hbm.at[pl.ds(base + ci * CHUNK, CHUNK)],
                    sem_s,
                )

        pltpu.make_async_copy(
            rows_v.at[1],
            out_hbm.at[pl.ds(base + (nchunks - 1) * CHUNK, CHUNK)],
            sem_s,
        ).wait()

    return emb


def kernel(x, lut):
    B = x.shape[0] * x.shape[1]
    out = _make_emb(B)(x.reshape(B), lut)
    return out.reshape(x.shape + (D_MODEL,))
